# baseline (speedup 1.0000x reference)
import functools

import jax
import jax.numpy as jnp
from jax import lax
from jax.experimental import pallas as pl
from jax.experimental.pallas import tpu as pltpu
from jax.experimental.pallas import tpu_sc as plsc

_VOCAB_PAD = 1024
_NCLS = 7
_CPAD = 8
_NW = 32
_GROUP = 16
_UNROLL = 8


def _prep_body(w_ref, table_ref, b_ref, inv_ref, out_ref):
    m = lax.dot_general(
        w_ref[...], table_ref[...], (((1,), (1,)), ((), ())),
        preferred_element_type=jnp.float32)
    out_ref[...] = (m + b_ref[:, :1]) * inv_ref[0]


def _make_sc_kernel(batch, seq):
    rows_per = batch // _NW
    n_groups = rows_per // _GROUP
    mesh = plsc.VectorSubcoreMesh(core_axis_name="c", subcore_axis_name="s")

    @functools.partial(
        pl.kernel,
        out_type=jax.ShapeDtypeStruct((batch, _NCLS), jnp.float32),
        mesh=mesh,
        scratch_types=[
            pltpu.VMEM((_CPAD * _VOCAB_PAD,), jnp.float32),
            pltpu.VMEM((rows_per, seq), jnp.int32),
            pltpu.VMEM((rows_per, _NCLS), jnp.float32),
        ],
        compiler_params=pltpu.CompilerParams(
            needs_layout_passes=False, use_tc_tiling_on_sc=False),
    )
    def sc_kernel(mt_hbm, x_hbm, out_hbm, mt_v, x_v, out_v):
        wid = lax.axis_index("s") * 2 + lax.axis_index("c")
        base = wid * rows_per
        pltpu.sync_copy(mt_hbm, mt_v)
        pltpu.sync_copy(x_hbm.at[pl.ds(base, rows_per)], x_v)

        lanes = jnp.arange(_GROUP, dtype=jnp.int32)

        def group_body(g, _):
            rows = g * _GROUP + lanes

            def seq_body(t, accs):
                idxs = []
                for u in range(_UNROLL):
                    col = jnp.zeros((_GROUP,), jnp.int32) + (t * _UNROLL + u)
                    idxs.append(plsc.load_gather(x_v, [rows, col]))
                accs = list(accs)
                for c in range(_NCLS):
                    vals = [plsc.load_gather(mt_v, [idx + (c * _VOCAB_PAD)])
                            for idx in idxs]
                    while len(vals) > 1:
                        vals = [vals[i] + vals[i + 1]
                                for i in range(0, len(vals) - 1, 2)] + (
                            [vals[-1]] if len(vals) % 2 else [])
                    accs[c] = accs[c] + vals[0]
                return tuple(accs)

            zero = jnp.zeros((_GROUP,), jnp.float32)
            accs = lax.fori_loop(0, seq // _UNROLL, seq_body,
                                 (zero,) * _NCLS)

            m = accs[0]
            for c in range(1, _NCLS):
                m = jnp.maximum(m, accs[c])
            es = [jnp.exp(a - m) for a in accs]
            tot = es[0]
            for c in range(1, _NCLS):
                tot = tot + es[c]
            for c in range(_NCLS):
                cvec = jnp.full((_GROUP,), c, jnp.int32)
                plsc.store_scatter(out_v, [rows, cvec], es[c] / tot)
            return 0

        lax.fori_loop(0, n_groups, group_body, 0)
        pltpu.sync_copy(out_v, out_hbm.at[pl.ds(base, rows_per)])

    return sc_kernel


def kernel(x, table, W, b):
    batch, seq = x.shape
    x = x.astype(jnp.int32)
    table_p = jnp.pad(table, ((0, _VOCAB_PAD - table.shape[0]), (0, 0)))
    w_p = jnp.pad(W, ((0, _CPAD - W.shape[0]), (0, 0)))
    b_p = jnp.broadcast_to(
        jnp.pad(b, (0, _CPAD - b.shape[0])).reshape(_CPAD, 1), (_CPAD, 128))
    inv = jnp.full((1,), 1.0 / seq, jnp.float32)

    mt = pl.pallas_call(
        _prep_body,
        out_shape=jax.ShapeDtypeStruct((_CPAD, _VOCAB_PAD), jnp.float32),
        in_specs=[
            pl.BlockSpec(memory_space=pltpu.VMEM),
            pl.BlockSpec(memory_space=pltpu.VMEM),
            pl.BlockSpec(memory_space=pltpu.VMEM),
            pl.BlockSpec(memory_space=pltpu.SMEM),
        ],
        out_specs=pl.BlockSpec(memory_space=pltpu.VMEM),
    )(w_p, table_p, b_p, inv)

    return _make_sc_kernel(batch, seq)(
        mt.reshape(_CPAD * _VOCAB_PAD), x)


# speedup vs baseline: 1.7267x; 1.7267x over previous
import functools

import jax
import jax.numpy as jnp
from jax import lax
from jax.experimental import pallas as pl
from jax.experimental.pallas import tpu as pltpu
from jax.experimental.pallas import tpu_sc as plsc

_VOCAB_PAD = 1024
_NCLS = 7
_CPAD = 8
_NPAIR = 4
_NW = 32
_GROUP = 16
_UNROLL = 8


def _prep_body(w_ref, table_ref, b_ref, inv_ref, out_ref):
    m = lax.dot_general(
        w_ref[...], table_ref[...], (((1,), (1,)), ((), ())),
        preferred_element_type=jnp.float32)
    m = (m + b_ref[:, :1]) * inv_ref[0]
    mu = lax.bitcast_convert_type(
        m.astype(jnp.bfloat16), jnp.uint16).astype(jnp.uint32)
    lo = lax.slice(mu, (0, 0), (_NPAIR, _VOCAB_PAD))
    hi = lax.slice(mu, (_NPAIR, 0), (_CPAD, _VOCAB_PAD))
    out_ref[...] = lax.bitcast_convert_type(
        lo | (hi << jnp.uint32(16)), jnp.int32)


def _make_sc_kernel(batch, seq):
    rows_per = batch // _NW
    n_groups = rows_per // _GROUP
    mesh = plsc.VectorSubcoreMesh(core_axis_name="c", subcore_axis_name="s")
    hi_mask = jnp.int32(-65536)

    @functools.partial(
        pl.kernel,
        out_type=jax.ShapeDtypeStruct((_NCLS, batch), jnp.float32),
        mesh=mesh,
        scratch_types=[
            pltpu.VMEM((_VOCAB_PAD,), jnp.int32),
            pltpu.VMEM((_VOCAB_PAD,), jnp.int32),
            pltpu.VMEM((_VOCAB_PAD,), jnp.int32),
            pltpu.VMEM((_VOCAB_PAD,), jnp.int32),
            pltpu.VMEM((seq, rows_per), jnp.int32),
            pltpu.VMEM((_NCLS, rows_per), jnp.float32),
        ],
        compiler_params=pltpu.CompilerParams(
            needs_layout_passes=False, use_tc_tiling_on_sc=False),
    )
    def sc_kernel(mtp_hbm, xt_hbm, out_hbm, m01, m23, m45, m67, x_v, out_v):
        wid = lax.axis_index("s") * 2 + lax.axis_index("c")
        base = wid * rows_per
        for p, mv in enumerate((m01, m23, m45, m67)):
            pltpu.sync_copy(mtp_hbm.at[p], mv)
        pltpu.sync_copy(xt_hbm.at[:, pl.ds(base, rows_per)], x_v)
        mrefs = (m01, m23, m45, m67)

        def group_body(g, _):
            col0 = g * _GROUP

            def seq_body(t, accs):
                accs = list(accs)
                for u in range(_UNROLL):
                    j = t * _UNROLL + u
                    idx = x_v[j, pl.ds(col0, _GROUP)]
                    for p in range(_NPAIR):
                        pair = plsc.load_gather(mrefs[p], [idx])
                        accs[2 * p] = accs[2 * p] + plsc.bitcast(
                            pair << 16, jnp.float32)
                        if 2 * p + 1 < _NCLS:
                            accs[2 * p + 1] = accs[2 * p + 1] + plsc.bitcast(
                                pair & hi_mask, jnp.float32)
                return tuple(accs)

            zero = jnp.zeros((_GROUP,), jnp.float32)
            accs = lax.fori_loop(0, seq // _UNROLL, seq_body,
                                 (zero,) * _NCLS)

            m = accs[0]
            for c in range(1, _NCLS):
                m = jnp.maximum(m, accs[c])
            es = [jnp.exp(a - m) for a in accs]
            tot = es[0]
            for c in range(1, _NCLS):
                tot = tot + es[c]
            for c in range(_NCLS):
                out_v[c, pl.ds(col0, _GROUP)] = es[c] / tot
            return 0

        lax.fori_loop(0, n_groups, group_body, 0)
        pltpu.sync_copy(out_v, out_hbm.at[:, pl.ds(base, rows_per)])

    return sc_kernel


def kernel(x, table, W, b):
    batch, seq = x.shape
    x = x.astype(jnp.int32)
    perm = [0, 2, 4, 6, 1, 3, 5, 7]
    table_p = jnp.pad(table, ((0, _VOCAB_PAD - table.shape[0]), (0, 0)))
    w_p = jnp.pad(W, ((0, _CPAD - W.shape[0]), (0, 0)))[jnp.array(perm)]
    b_p = jnp.broadcast_to(
        jnp.pad(b, (0, _CPAD - b.shape[0]))[jnp.array(perm)].reshape(
            _CPAD, 1), (_CPAD, 128))
    inv = jnp.full((1,), 1.0 / seq, jnp.float32)

    mtp = pl.pallas_call(
        _prep_body,
        out_shape=jax.ShapeDtypeStruct((_NPAIR, _VOCAB_PAD), jnp.int32),
        in_specs=[
            pl.BlockSpec(memory_space=pltpu.VMEM),
            pl.BlockSpec(memory_space=pltpu.VMEM),
            pl.BlockSpec(memory_space=pltpu.VMEM),
            pl.BlockSpec(memory_space=pltpu.SMEM),
        ],
        out_specs=pl.BlockSpec(memory_space=pltpu.VMEM),
    )(w_p, table_p, b_p, inv)

    out_t = _make_sc_kernel(batch, seq)(mtp, x.T)
    return out_t.T


# speedup vs baseline: 1.7273x; 1.0004x over previous
import functools

import jax
import jax.numpy as jnp
from jax import lax
from jax.experimental import pallas as pl
from jax.experimental.pallas import tpu as pltpu
from jax.experimental.pallas import tpu_sc as plsc

_VOCAB_PAD = 1024
_NCLS = 7
_CPAD = 8
_NPAIR = 4
_NW = 32
_GROUP = 16
_UNROLL = 8


def _prep_body(w_ref, table_ref, b_ref, inv_ref, out_ref):
    m = lax.dot_general(
        w_ref[...], table_ref[...], (((1,), (1,)), ((), ())),
        preferred_element_type=jnp.float32)
    m = (m + b_ref[:, :1]) * inv_ref[0]
    mu = lax.bitcast_convert_type(
        m.astype(jnp.bfloat16), jnp.uint16).astype(jnp.uint32)
    lo = lax.slice(mu, (0, 0), (_NPAIR, _VOCAB_PAD))
    hi = lax.slice(mu, (_NPAIR, 0), (_CPAD, _VOCAB_PAD))
    out_ref[...] = lax.bitcast_convert_type(
        lo | (hi << jnp.uint32(16)), jnp.int32)


def _make_sc_kernel(batch, seq):
    rows_per = batch // _NW
    n_groups = rows_per // _GROUP
    mesh = plsc.VectorSubcoreMesh(core_axis_name="c", subcore_axis_name="s")
    hi_mask = jnp.int32(-65536)

    @functools.partial(
        pl.kernel,
        out_type=jax.ShapeDtypeStruct((_NCLS, batch), jnp.float32),
        mesh=mesh,
        scratch_types=[
            pltpu.VMEM((_VOCAB_PAD,), jnp.int32),
            pltpu.VMEM((_VOCAB_PAD,), jnp.int32),
            pltpu.VMEM((_VOCAB_PAD,), jnp.int32),
            pltpu.VMEM((_VOCAB_PAD,), jnp.int32),
            pltpu.VMEM((seq, rows_per), jnp.int32),
            pltpu.VMEM((_NCLS, rows_per), jnp.float32),
        ],
        compiler_params=pltpu.CompilerParams(
            needs_layout_passes=False, use_tc_tiling_on_sc=False),
    )
    def sc_kernel(mtp_hbm, xt_hbm, out_hbm, m01, m23, m45, m67, x_v, out_v):
        wid = lax.axis_index("s") * 2 + lax.axis_index("c")
        base = wid * rows_per
        for p, mv in enumerate((m01, m23, m45, m67)):
            pltpu.sync_copy(mtp_hbm.at[p], mv)
        pltpu.sync_copy(xt_hbm.at[:, pl.ds(base, rows_per)], x_v)
        mrefs = (m01, m23, m45, m67)

        zero = jnp.zeros((_GROUP,), jnp.float32)

        @plsc.parallel_loop(0, n_groups, 1)
        def group_body(g):
            col0 = g * _GROUP

            @plsc.parallel_loop(0, seq // _UNROLL, 1,
                                carry=(zero,) * _NCLS)
            def seq_body(t, accs):
                accs = list(accs)
                for u in range(_UNROLL):
                    j = t * _UNROLL + u
                    idx = x_v[j, pl.ds(col0, _GROUP)]
                    for p in range(_NPAIR):
                        pair = plsc.load_gather(mrefs[p], [idx])
                        accs[2 * p] = accs[2 * p] + plsc.bitcast(
                            pair << 16, jnp.float32)
                        if 2 * p + 1 < _NCLS:
                            accs[2 * p + 1] = accs[2 * p + 1] + plsc.bitcast(
                                pair & hi_mask, jnp.float32)
                return tuple(accs)

            accs = seq_body
            m = accs[0]
            for c in range(1, _NCLS):
                m = jnp.maximum(m, accs[c])
            es = [jnp.exp(a - m) for a in accs]
            tot = es[0]
            for c in range(1, _NCLS):
                tot = tot + es[c]
            for c in range(_NCLS):
                out_v[c, pl.ds(col0, _GROUP)] = es[c] / tot
        pltpu.sync_copy(out_v, out_hbm.at[:, pl.ds(base, rows_per)])

    return sc_kernel


def kernel(x, table, W, b):
    batch, seq = x.shape
    x = x.astype(jnp.int32)
    perm = [0, 2, 4, 6, 1, 3, 5, 7]
    table_p = jnp.pad(table, ((0, _VOCAB_PAD - table.shape[0]), (0, 0)))
    w_p = jnp.pad(W, ((0, _CPAD - W.shape[0]), (0, 0)))[jnp.array(perm)]
    b_p = jnp.broadcast_to(
        jnp.pad(b, (0, _CPAD - b.shape[0]))[jnp.array(perm)].reshape(
            _CPAD, 1), (_CPAD, 128))
    inv = jnp.full((1,), 1.0 / seq, jnp.float32)

    mtp = pl.pallas_call(
        _prep_body,
        out_shape=jax.ShapeDtypeStruct((_NPAIR, _VOCAB_PAD), jnp.int32),
        in_specs=[
            pl.BlockSpec(memory_space=pltpu.VMEM),
            pl.BlockSpec(memory_space=pltpu.VMEM),
            pl.BlockSpec(memory_space=pltpu.VMEM),
            pl.BlockSpec(memory_space=pltpu.SMEM),
        ],
        out_specs=pl.BlockSpec(memory_space=pltpu.VMEM),
    )(w_p, table_p, b_p, inv)

    out_t = _make_sc_kernel(batch, seq)(mtp, x.T)
    return out_t.T


# speedup vs baseline: 1.7435x; 1.0094x over previous
import functools

import jax
import jax.numpy as jnp
from jax import lax
from jax.experimental import pallas as pl
from jax.experimental.pallas import tpu as pltpu
from jax.experimental.pallas import tpu_sc as plsc

_VOCAB_PAD = 1024
_NCLS = 7
_CPAD = 8
_NPAIR = 4
_NW = 32
_GROUP = 16
_UNROLL = 8


def _prep_body(w_ref, table_ref, b_ref, inv_ref, out_ref):
    m = lax.dot_general(
        w_ref[...], table_ref[...], (((1,), (1,)), ((), ())),
        preferred_element_type=jnp.float32)
    m = (m + b_ref[:, :1]) * inv_ref[0]
    mu = lax.bitcast_convert_type(
        m.astype(jnp.bfloat16), jnp.uint16).astype(jnp.uint32)
    lo = lax.slice(mu, (0, 0), (_NPAIR, _VOCAB_PAD))
    hi = lax.slice(mu, (_NPAIR, 0), (_CPAD, _VOCAB_PAD))
    out_ref[...] = lax.bitcast_convert_type(
        lo | (hi << jnp.uint32(16)), jnp.int32)


def _make_sc_kernel(batch, seq):
    rows_per = batch // _NW
    n_groups = rows_per // _GROUP
    mesh = plsc.VectorSubcoreMesh(core_axis_name="c", subcore_axis_name="s")
    hi_mask = jnp.int32(-65536)

    @functools.partial(
        pl.kernel,
        out_type=jax.ShapeDtypeStruct((_NCLS, batch), jnp.float32),
        mesh=mesh,
        scratch_types=[
            pltpu.VMEM((_VOCAB_PAD,), jnp.int32),
            pltpu.VMEM((_VOCAB_PAD,), jnp.int32),
            pltpu.VMEM((_VOCAB_PAD,), jnp.int32),
            pltpu.VMEM((_VOCAB_PAD,), jnp.int32),
            pltpu.VMEM((seq, rows_per), jnp.int32),
            pltpu.VMEM((_NCLS, rows_per), jnp.float32),
            pltpu.SemaphoreType.DMA,
            pltpu.SemaphoreType.DMA,
            pltpu.SemaphoreType.DMA,
            pltpu.SemaphoreType.DMA,
        ],
        compiler_params=pltpu.CompilerParams(
            needs_layout_passes=False, use_tc_tiling_on_sc=False),
    )
    def sc_kernel(mtp_hbm, xt_hbm, out_hbm, m01, m23, m45, m67, x_v, out_v,
                  s0, s1, s2, s3):
        wid = lax.axis_index("s") * 2 + lax.axis_index("c")
        base = wid * rows_per
        nch = 4
        cw = rows_per // nch
        sems = (s0, s1, s2, s3)
        copies = []
        for k in range(nch):
            copies.append(pltpu.async_copy(
                xt_hbm.at[:, pl.ds(base + k * cw, cw)],
                x_v.at[:, pl.ds(k * cw, cw)], sems[k]))
        for p, mv in enumerate((m01, m23, m45, m67)):
            pltpu.sync_copy(mtp_hbm.at[p], mv)
        mrefs = (m01, m23, m45, m67)

        zero = jnp.zeros((_GROUP,), jnp.float32)
        gp_per_ch = n_groups // nch

        def chunk_groups(k):
            copies[k].wait()

            @plsc.parallel_loop(k * gp_per_ch, (k + 1) * gp_per_ch, 1)
            def group_body(g):
                _do_group(g)

        def _do_group(g):
            col0 = g * _GROUP

            @plsc.parallel_loop(0, seq // _UNROLL, 1,
                                carry=(zero,) * _NCLS)
            def seq_body(t, accs):
                accs = list(accs)
                for u in range(_UNROLL):
                    j = t * _UNROLL + u
                    idx = x_v[j, pl.ds(col0, _GROUP)]
                    for p in range(_NPAIR):
                        pair = plsc.load_gather(mrefs[p], [idx])
                        accs[2 * p] = accs[2 * p] + plsc.bitcast(
                            pair << 16, jnp.float32)
                        if 2 * p + 1 < _NCLS:
                            accs[2 * p + 1] = accs[2 * p + 1] + plsc.bitcast(
                                pair & hi_mask, jnp.float32)
                return tuple(accs)

            accs = seq_body
            m = accs[0]
            for c in range(1, _NCLS):
                m = jnp.maximum(m, accs[c])
            es = [jnp.exp(a - m) for a in accs]
            tot = es[0]
            for c in range(1, _NCLS):
                tot = tot + es[c]
            for c in range(_NCLS):
                out_v[c, pl.ds(col0, _GROUP)] = es[c] / tot

        for k in range(nch):
            chunk_groups(k)
        pltpu.sync_copy(out_v, out_hbm.at[:, pl.ds(base, rows_per)])

    return sc_kernel


def kernel(x, table, W, b):
    batch, seq = x.shape
    x = x.astype(jnp.int32)
    perm = [0, 2, 4, 6, 1, 3, 5, 7]
    table_p = jnp.pad(table, ((0, _VOCAB_PAD - table.shape[0]), (0, 0)))
    w_p = jnp.pad(W, ((0, _CPAD - W.shape[0]), (0, 0)))[jnp.array(perm)]
    b_p = jnp.broadcast_to(
        jnp.pad(b, (0, _CPAD - b.shape[0]))[jnp.array(perm)].reshape(
            _CPAD, 1), (_CPAD, 128))
    inv = jnp.full((1,), 1.0 / seq, jnp.float32)

    mtp = pl.pallas_call(
        _prep_body,
        out_shape=jax.ShapeDtypeStruct((_NPAIR, _VOCAB_PAD), jnp.int32),
        in_specs=[
            pl.BlockSpec(memory_space=pltpu.VMEM),
            pl.BlockSpec(memory_space=pltpu.VMEM),
            pl.BlockSpec(memory_space=pltpu.VMEM),
            pl.BlockSpec(memory_space=pltpu.SMEM),
        ],
        out_specs=pl.BlockSpec(memory_space=pltpu.VMEM),
    )(w_p, table_p, b_p, inv)

    out_t = _make_sc_kernel(batch, seq)(mtp, x.T)
    return out_t.T
